# baseline (speedup 1.0000x reference)
import functools

import jax
import jax.numpy as jnp
from jax import lax
from jax.experimental import pallas as pl
from jax.experimental.pallas import tpu as pltpu
from jax.experimental.pallas import tpu_sc as plsc

_B, _L = 16, 4096
_PAGE = 64
_P = _L // _PAGE
_LANES = 16
_TR, _TC = 8, 128
_NTR, _NTC = _B // _TR, _L // _TC
_NW = 32
_TILES_W = (_NTR * _NTC) // _NW
_PG_T = _TC // _PAGE
_NPAGES = _B * _P
_MEAN_W, _MAX_W = 0.7, 0.3
_NEG = -1e9


def _seg_body(mask_hbm, score_hbm, t2p_hbm, ps_hbm, pv_hbm,
              mask_v, score_v, t2p_v, ps_v, pv_v, in_sems, out_sems):
    wid = lax.axis_index("s") * 2 + lax.axis_index("c")

    lane = lax.iota(jnp.int32, _LANES)
    row_vec = lane >> 1
    pg_vec = lane & 1
    neg1 = jnp.full((_LANES,), -1, jnp.int32)
    zero_f = jnp.zeros((_LANES,), jnp.float32)
    one_i = jnp.full((_LANES,), 1, jnp.int32)
    zero_i = jnp.zeros((_LANES,), jnp.int32)
    m63 = jnp.full((_LANES,), _PAGE - 1, jnp.int32)
    col_base = pg_vec * _PAGE

    tiles = []
    in_copies = []
    for k in range(_TILES_W):
        t = wid * _TILES_W + k
        tr = t // _NTC
        tc = t % _NTC
        r0 = tr * _TR
        c0 = tc * _TC
        tiles.append((tr, tc, r0, c0))
        in_copies.append((
            pltpu.async_copy(mask_hbm.at[pl.ds(r0, _TR), pl.ds(c0, _TC)],
                             mask_v.at[k], in_sems.at[2 * k]),
            pltpu.async_copy(score_hbm.at[pl.ds(r0, _TR), pl.ds(c0, _TC)],
                             score_v.at[k], in_sems.at[2 * k + 1]),
        ))

    out_copies = []
    for k in range(_TILES_W):
        tr, tc, r0, c0 = tiles[k]
        for c in in_copies[k]:
            c.wait()

        page_vec = pg_vec + tc * _PG_T

        def body(j, carry, k=k, page_vec=page_vec):
            s, mx, cnt = carry
            col = col_base + ((lane + j) & m63)
            sc = plsc.load_gather(score_v.at[k], [row_vec, col])
            mk = plsc.load_gather(mask_v.at[k], [row_vec, col])
            valid = mk != 0
            plsc.store_scatter(t2p_v.at[k], [row_vec, col],
                               jnp.where(valid, page_vec, neg1))
            s = s + jnp.where(valid, sc, zero_f)
            mx = jnp.maximum(mx, jnp.where(valid, sc, _NEG))
            cnt = cnt + jnp.where(valid, one_i, zero_i)
            return (s, mx, cnt)

        s0 = jnp.zeros((_LANES,), jnp.float32)
        mx0 = jnp.full((_LANES,), _NEG, jnp.float32)
        c0i = jnp.zeros((_LANES,), jnp.int32)
        s, mx, cnt = lax.fori_loop(0, _PAGE, body, (s0, mx0, c0i), unroll=8)

        out_copies.append(
            pltpu.async_copy(t2p_v.at[k],
                             t2p_hbm.at[pl.ds(r0, _TR), pl.ds(c0, _TC)],
                             out_sems.at[k]))

        cntf = jnp.maximum(cnt, 1).astype(jnp.float32)
        raw = _MEAN_W * (s / cntf) + _MAX_W * mx
        valid_page = cnt > 0
        ps_v[pl.ds(k * _LANES, _LANES)] = jnp.where(valid_page, raw, zero_f)
        pv_v[pl.ds(k * _LANES, _LANES)] = jnp.where(valid_page, one_i, zero_i)

    pltpu.sync_copy(ps_v, ps_hbm.at[pl.ds(wid * 2 * _LANES, 2 * _LANES)])
    pltpu.sync_copy(pv_v, pv_hbm.at[pl.ds(wid * 2 * _LANES, 2 * _LANES)])
    for c in out_copies:
        c.wait()


@functools.lru_cache(maxsize=1)
def _build_seg_kernel():
    return functools.partial(
        pl.kernel,
        out_type=(
            jax.ShapeDtypeStruct((_B, _L), jnp.int32),
            jax.ShapeDtypeStruct((_NPAGES,), jnp.float32),
            jax.ShapeDtypeStruct((_NPAGES,), jnp.int32),
        ),
        mesh=plsc.VectorSubcoreMesh(core_axis_name="c", subcore_axis_name="s"),
        compiler_params=pltpu.CompilerParams(
            needs_layout_passes=False,
            disable_bounds_checks=True,
            disable_semaphore_checks=True,
            use_tc_tiling_on_sc=True,
        ),
        scratch_types=[
            pltpu.VMEM((_TILES_W, _TR, _TC), jnp.int32),
            pltpu.VMEM((_TILES_W, _TR, _TC), jnp.float32),
            pltpu.VMEM((_TILES_W, _TR, _TC), jnp.int32),
            pltpu.VMEM((2 * _LANES,), jnp.float32),
            pltpu.VMEM((2 * _LANES,), jnp.int32),
            pltpu.SemaphoreType.DMA((2 * _TILES_W,)),
            pltpu.SemaphoreType.DMA((_TILES_W,)),
        ],
    )(_seg_body)


def _perm_idx():
    import numpy as np
    b = np.arange(_B)[:, None]
    p = np.arange(_P)[None, :]
    tr, r = b // _TR, b % _TR
    tc, pg = p // _PG_T, p % _PG_T
    return jnp.asarray(16 * (tr * _NTC + tc) + 2 * r + pg, dtype=jnp.int32)


def kernel(input_ids, attention_mask, token_scores):
    del input_ids
    t2p, ps, pv = _build_seg_kernel()(attention_mask, token_scores)
    idx = _perm_idx()
    return (ps[idx], t2p, pv[idx] != 0)


# speedup vs baseline: 1.3880x; 1.3880x over previous
import functools

import jax
import jax.numpy as jnp
from jax import lax
from jax.experimental import pallas as pl
from jax.experimental.pallas import tpu as pltpu
from jax.experimental.pallas import tpu_sc as plsc

_B, _L = 16, 4096
_PAGE = 64
_P = _L // _PAGE
_LANES = 16
_TR, _TC = 8, 128
_NTR, _NTC = _B // _TR, _L // _TC
_NW = 32
_TILES_W = (_NTR * _NTC) // _NW
_PG_T = _TC // _PAGE
_NPAGES = _B * _P
_MEAN_W, _MAX_W = 0.7, 0.3
_NEG = -1e9


def _seg_body(mask_hbm, score_hbm, t2p_hbm, ps_hbm, pv_hbm,
              mask_v, score_v, t2p_v, ps_v, pv_v, in_sems, out_sems):
    wid = lax.axis_index("s") * 2 + lax.axis_index("c")

    lane = lax.iota(jnp.int32, _LANES)
    row_vec = lane >> 1
    pg_vec = lane & 1
    neg1 = jnp.full((_LANES,), -1, jnp.int32)
    zero_f = jnp.zeros((_LANES,), jnp.float32)
    one_i = jnp.full((_LANES,), 1, jnp.int32)
    zero_i = jnp.zeros((_LANES,), jnp.int32)
    m63 = jnp.full((_LANES,), _PAGE - 1, jnp.int32)
    col_base = pg_vec * _PAGE

    tr = wid // (_NW // _NTR)
    q = wid % (_NW // _NTR)
    r0 = tr * _TR
    c0 = q * (_TILES_W * _TC)
    in0 = pltpu.async_copy(
        mask_hbm.at[pl.ds(r0, _TR), pl.ds(c0, _TILES_W * _TC)],
        mask_v, in_sems.at[0])
    in1 = pltpu.async_copy(
        score_hbm.at[pl.ds(r0, _TR), pl.ds(c0, _TILES_W * _TC)],
        score_v, in_sems.at[1])
    in0.wait()
    in1.wait()

    for k in range(_TILES_W):
        tc = q * _TILES_W + k
        page_vec = pg_vec + tc * _PG_T
        tile_col = col_base + k * _TC

        def body(j, carry, tile_col=tile_col, page_vec=page_vec):
            s, mx, cnt = carry
            col = tile_col + ((lane + j) & m63)
            sc = plsc.load_gather(score_v, [row_vec, col])
            mk = plsc.load_gather(mask_v, [row_vec, col])
            valid = mk != 0
            plsc.store_scatter(t2p_v, [row_vec, col],
                               jnp.where(valid, page_vec, neg1))
            s = s + jnp.where(valid, sc, zero_f)
            mx = jnp.maximum(mx, jnp.where(valid, sc, _NEG))
            cnt = cnt + jnp.where(valid, one_i, zero_i)
            return (s, mx, cnt)

        s0 = jnp.zeros((_LANES,), jnp.float32)
        mx0 = jnp.full((_LANES,), _NEG, jnp.float32)
        c0i = jnp.zeros((_LANES,), jnp.int32)
        s, mx, cnt = lax.fori_loop(0, _PAGE, body, (s0, mx0, c0i), unroll=8)

        cntf = jnp.maximum(cnt, 1).astype(jnp.float32)
        raw = _MEAN_W * (s / cntf) + _MAX_W * mx
        valid_page = cnt > 0
        ps_v[pl.ds(k * _LANES, _LANES)] = jnp.where(valid_page, raw, zero_f)
        pv_v[pl.ds(k * _LANES, _LANES)] = jnp.where(valid_page, one_i, zero_i)

    out0 = pltpu.async_copy(
        t2p_v, t2p_hbm.at[pl.ds(r0, _TR), pl.ds(c0, _TILES_W * _TC)],
        out_sems.at[0])
    pltpu.sync_copy(ps_v, ps_hbm.at[pl.ds(wid * 2 * _LANES, 2 * _LANES)])
    pltpu.sync_copy(pv_v, pv_hbm.at[pl.ds(wid * 2 * _LANES, 2 * _LANES)])
    out0.wait()


@functools.lru_cache(maxsize=1)
def _build_seg_kernel():
    return functools.partial(
        pl.kernel,
        out_type=(
            jax.ShapeDtypeStruct((_B, _L), jnp.int32),
            jax.ShapeDtypeStruct((_NPAGES,), jnp.float32),
            jax.ShapeDtypeStruct((_NPAGES,), jnp.int32),
        ),
        mesh=plsc.VectorSubcoreMesh(core_axis_name="c", subcore_axis_name="s"),
        compiler_params=pltpu.CompilerParams(
            needs_layout_passes=False,
            disable_bounds_checks=True,
            disable_semaphore_checks=True,
            use_tc_tiling_on_sc=True,
        ),
        scratch_types=[
            pltpu.VMEM((_TR, _TILES_W * _TC), jnp.int32),
            pltpu.VMEM((_TR, _TILES_W * _TC), jnp.float32),
            pltpu.VMEM((_TR, _TILES_W * _TC), jnp.int32),
            pltpu.VMEM((2 * _LANES,), jnp.float32),
            pltpu.VMEM((2 * _LANES,), jnp.int32),
            pltpu.SemaphoreType.DMA((2,)),
            pltpu.SemaphoreType.DMA((1,)),
        ],
    )(_seg_body)


def _unpermute(flat):
    return (flat.reshape(_NTR, _NTC, _TR, _PG_T)
                .transpose(0, 2, 1, 3)
                .reshape(_B, _P))


def kernel(input_ids, attention_mask, token_scores):
    del input_ids
    t2p, ps, pv = _build_seg_kernel()(attention_mask, token_scores)
    return (_unpermute(ps), t2p, _unpermute(pv).astype(bool))


# speedup vs baseline: 1.4378x; 1.0358x over previous
import functools

import jax
import jax.numpy as jnp
from jax import lax
from jax.experimental import pallas as pl
from jax.experimental.pallas import tpu as pltpu
from jax.experimental.pallas import tpu_sc as plsc

_B, _L = 16, 4096
_PAGE = 64
_P = _L // _PAGE
_LANES = 16
_TR, _TC = 8, 128
_NTR, _NTC = _B // _TR, _L // _TC
_NW = 32
_TILES_W = (_NTR * _NTC) // _NW
_PG_T = _TC // _PAGE
_NPAGES = _B * _P
_MEAN_W, _MAX_W = 0.7, 0.3
_NEG = -1e9


def _seg_body(mask_hbm, score_hbm, t2p_hbm, ps_hbm, pv_hbm,
              mask_v, score_v, t2p_v, ps_v, pv_v, in_sems, out_sems):
    wid = lax.axis_index("s") * 2 + lax.axis_index("c")

    lane = lax.iota(jnp.int32, _LANES)
    row_vec = lane >> 1
    pg_vec = lane & 1
    neg1 = jnp.full((_LANES,), -1, jnp.int32)
    zero_f = jnp.zeros((_LANES,), jnp.float32)
    one_i = jnp.full((_LANES,), 1, jnp.int32)
    zero_i = jnp.zeros((_LANES,), jnp.int32)
    m63 = jnp.full((_LANES,), _PAGE - 1, jnp.int32)
    col_base = pg_vec * _PAGE

    tr = wid // (_NW // _NTR)
    q = wid % (_NW // _NTR)
    r0 = tr * _TR
    c0 = q * (_TILES_W * _TC)
    in0 = pltpu.async_copy(
        mask_hbm.at[pl.ds(r0, _TR), pl.ds(c0, _TILES_W * _TC)],
        mask_v, in_sems.at[0])
    in1 = pltpu.async_copy(
        score_hbm.at[pl.ds(r0, _TR), pl.ds(c0, _TILES_W * _TC)],
        score_v, in_sems.at[1])
    in0.wait()
    in1.wait()

    for k in range(_TILES_W):
        tc = q * _TILES_W + k
        page_vec = pg_vec + tc * _PG_T
        tile_col = col_base + k * _TC

        def body(j, carry, tile_col=tile_col, page_vec=page_vec):
            s, mx, cnt = carry
            col = tile_col + ((lane + j) & m63)
            sc = plsc.load_gather(score_v, [row_vec, col])
            mk = plsc.load_gather(mask_v, [row_vec, col])
            valid = mk != 0
            plsc.store_scatter(t2p_v, [row_vec, col],
                               jnp.where(valid, page_vec, neg1))
            s = s + jnp.where(valid, sc, zero_f)
            mx = jnp.maximum(mx, jnp.where(valid, sc, _NEG))
            cnt = cnt + jnp.where(valid, one_i, zero_i)
            return (s, mx, cnt)

        s0 = jnp.zeros((_LANES,), jnp.float32)
        mx0 = jnp.full((_LANES,), _NEG, jnp.float32)
        c0i = jnp.zeros((_LANES,), jnp.int32)
        s, mx, cnt = lax.fori_loop(0, _PAGE, body, (s0, mx0, c0i), unroll=2)

        cntf = jnp.maximum(cnt, 1).astype(jnp.float32)
        raw = _MEAN_W * (s / cntf) + _MAX_W * mx
        valid_page = cnt > 0
        ps_v[pl.ds(k * _LANES, _LANES)] = jnp.where(valid_page, raw, zero_f)
        pv_v[pl.ds(k * _LANES, _LANES)] = jnp.where(valid_page, one_i, zero_i)

    out0 = pltpu.async_copy(
        t2p_v, t2p_hbm.at[pl.ds(r0, _TR), pl.ds(c0, _TILES_W * _TC)],
        out_sems.at[0])
    pltpu.sync_copy(ps_v, ps_hbm.at[pl.ds(wid * 2 * _LANES, 2 * _LANES)])
    pltpu.sync_copy(pv_v, pv_hbm.at[pl.ds(wid * 2 * _LANES, 2 * _LANES)])
    out0.wait()


@functools.lru_cache(maxsize=1)
def _build_seg_kernel():
    return functools.partial(
        pl.kernel,
        out_type=(
            jax.ShapeDtypeStruct((_B, _L), jnp.int32),
            jax.ShapeDtypeStruct((_NPAGES,), jnp.float32),
            jax.ShapeDtypeStruct((_NPAGES,), jnp.int32),
        ),
        mesh=plsc.VectorSubcoreMesh(core_axis_name="c", subcore_axis_name="s"),
        compiler_params=pltpu.CompilerParams(
            needs_layout_passes=False,
            disable_bounds_checks=True,
            disable_semaphore_checks=True,
            use_tc_tiling_on_sc=True,
        ),
        scratch_types=[
            pltpu.VMEM((_TR, _TILES_W * _TC), jnp.int32),
            pltpu.VMEM((_TR, _TILES_W * _TC), jnp.float32),
            pltpu.VMEM((_TR, _TILES_W * _TC), jnp.int32),
            pltpu.VMEM((2 * _LANES,), jnp.float32),
            pltpu.VMEM((2 * _LANES,), jnp.int32),
            pltpu.SemaphoreType.DMA((2,)),
            pltpu.SemaphoreType.DMA((1,)),
        ],
    )(_seg_body)


def _unpermute(flat):
    return (flat.reshape(_NTR, _NTC, _TR, _PG_T)
                .transpose(0, 2, 1, 3)
                .reshape(_B, _P))


def kernel(input_ids, attention_mask, token_scores):
    del input_ids
    t2p, ps, pv = _build_seg_kernel()(attention_mask, token_scores)
    return (_unpermute(ps), t2p, _unpermute(pv).astype(bool))
